# baseline (speedup 1.0000x reference)
import functools

import jax
import jax.numpy as jnp
from jax import lax
from jax.experimental import pallas as pl
from jax.experimental.pallas import tpu as pltpu
from jax.experimental.pallas import tpu_sc as plsc

D = 64
L = 16
NC, NS = 2, 16
NW = NC * NS
C = 128
DH, DL = D // 8, 8
NB = C // 128


def _lane_shuffle(v, idx):
    return lax.gather(
        v, idx[:, None],
        lax.GatherDimensionNumbers(
            offset_dims=(), collapsed_slice_dims=(0,), start_index_map=(0,)),
        (1,), mode=lax.GatherScatterMode.PROMISE_IN_BOUNDS)


def _rsqrt_vec(x):
    i = lax.bitcast_convert_type(x, jnp.int32)
    i = jnp.int32(0x5F3759DF) - lax.shift_right_arithmetic(i, jnp.int32(1))
    y = lax.bitcast_convert_type(i, jnp.float32)
    for _ in range(2):
        y = y * (jnp.float32(1.5) - jnp.float32(0.5) * x * y * y)
    return y


def _make_sc_call(seq, batch):
    n_rows = seq * batch
    n_chunks_total = n_rows // C
    assert n_chunks_total % (4 * NW) == 0
    chunks_per_w = n_chunks_total // NW
    bt_per_s = batch // C
    mesh = plsc.VectorSubcoreMesh(core_axis_name="c", subcore_axis_name="s")
    NBUF = 4

    @functools.partial(
        pl.kernel,
        mesh=mesh,
        compiler_params=pltpu.CompilerParams(use_tc_tiling_on_sc=False,
                                             needs_layout_passes=False),
        out_type=jax.ShapeDtypeStruct((seq, DH, batch // 128, DL, 128),
                                      jnp.float32),
        scratch_types=(
            [pltpu.VMEM((C,), jnp.int32)] * NBUF
            + [pltpu.VMEM((C, D), jnp.float32)] * NBUF
            + [pltpu.VMEM((DH, DL, 128), jnp.float32)] * NBUF
            + [pltpu.VMEM((D,), jnp.float32)] * 2
            + [pltpu.SemaphoreType.DMA] * (3 * NBUF)
        ),
    )
    def sc_fn(idx_hbm, table_hbm, gamma_hbm, beta_hbm, out_hbm, *bufs):
        wid = lax.axis_index("s") * NC + lax.axis_index("c")
        g_base = wid * chunks_per_w

        idx_v = bufs[0:NBUF]
        rows_v = bufs[NBUF:2 * NBUF]
        outt_v = bufs[2 * NBUF:3 * NBUF]
        gv, bv = bufs[3 * NBUF:3 * NBUF + 2]
        isem = bufs[3 * NBUF + 2:3 * NBUF + 2 + NBUF]
        gsem = bufs[3 * NBUF + 2 + NBUF:3 * NBUF + 2 + 2 * NBUF]
        osem = bufs[3 * NBUF + 2 + 2 * NBUF:3 * NBUF + 2 + 3 * NBUF]

        pltpu.sync_copy(gamma_hbm, gv)
        pltpu.sync_copy(beta_hbm, bv)
        gslab = [gv[pl.ds(j * L, L)] for j in range(D // L)]
        bslab = [bv[pl.ds(j * L, L)] for j in range(D // L)]

        def fire_idx(g, slot):
            pltpu.async_copy(idx_hbm.at[pl.ds(g * C, C)], idx_v[slot],
                             isem[slot])

        def wait_idx(slot):
            pltpu.make_async_copy(idx_hbm.at[pl.ds(0, C)], idx_v[slot],
                                  isem[slot]).wait()

        def fire_gather(slot):
            for j in range(C // L):
                sl = pl.ds(j * L, L)
                iv = lax.shift_left(idx_v[slot][sl], jnp.int32(1))
                pltpu.async_copy(table_hbm.at[iv], rows_v[slot].at[sl],
                                 gsem[slot])

        def wait_gather(slot):
            pltpu.make_async_copy(table_hbm.at[pl.ds(0, C)], rows_v[slot],
                                  gsem[slot]).wait()

        def fire_out(g, slot):
            s_i = g // bt_per_s
            bt_i = lax.rem(g, jnp.int32(bt_per_s))
            for dh in range(DH):
                pltpu.async_copy(outt_v[slot].at[dh],
                                 out_hbm.at[s_i, dh, bt_i], osem[slot])

        def wait_out(slot):
            pltpu.make_async_copy(outt_v[slot], out_hbm.at[0, :, 0],
                                  osem[slot]).wait()

        def compute_chunk(rows, outt):
            def group_body(gi, carry):
                rid = gi * L + lax.iota(jnp.int32, L)
                s = None
                ss = None
                for d in range(D):
                    cd = jnp.full((L,), d, jnp.int32)
                    v = plsc.load_gather(rows, [rid, cd])
                    s = v if s is None else s + v
                    ss = v * v if ss is None else ss + v * v
                mean = s * jnp.float32(1.0 / D)
                var = ss * jnp.float32(1.0 / D) - mean * mean
                scale = _rsqrt_vec(var + jnp.float32(1e-5))
                for d in range(D):
                    cd = jnp.full((L,), d, jnp.int32)
                    v = plsc.load_gather(rows, [rid, cd])
                    lane = jnp.full((L,), d % L, jnp.int32)
                    gd = _lane_shuffle(gslab[d // L], lane)
                    bd = _lane_shuffle(bslab[d // L], lane)
                    w = (v - mean) * scale * gd + bd
                    outt[d // DL, d % DL, pl.ds(gi * L, L)] = w
                return carry

            lax.fori_loop(0, C // L, group_body, 0)

        pltpu.sync_copy(idx_hbm.at[pl.ds(g_base * C, C)], idx_v[0])
        fire_gather(0)
        for k in range(1, NBUF):
            fire_idx(g_base + k, k)
        for k in range(1, NBUF - 1):
            wait_idx(k)
            fire_gather(k)

        def quad_body(p, carry):
            for off in range(NBUF):
                slot = off
                g = NBUF * p + off
                wait_gather(slot)

                @pl.when(g + NBUF < chunks_per_w)
                def _():
                    fire_idx(g_base + g + NBUF, slot)

                @pl.when(g >= NBUF)
                def _():
                    wait_out(slot)

                @pl.when(g + NBUF - 1 < chunks_per_w)
                def _():
                    pslot = (off + NBUF - 1) % NBUF
                    wait_idx(pslot)
                    fire_gather(pslot)

                compute_chunk(rows_v[slot], outt_v[slot])
                fire_out(g_base + g, slot)
            return carry

        lax.fori_loop(0, chunks_per_w // NBUF, quad_body, 0)
        for k in range(NBUF):
            wait_out(k)

    return sc_fn


def kernel(x, table, gamma, beta):
    b, s = x.shape
    v, d = table.shape
    assert d == D
    xt = x.T.reshape(-1).astype(jnp.int32)
    tablep = jnp.pad(table, ((0, 0), (0, d))).reshape(2 * v, d)
    out5 = _make_sc_call(s, b)(xt, tablep, gamma, beta)
    out = jnp.transpose(out5, (2, 4, 0, 1, 3)).reshape(b, s, d)
    return out


# speedup vs baseline: 1.8038x; 1.8038x over previous
import functools

import jax
import jax.numpy as jnp
from jax import lax
from jax.experimental import pallas as pl
from jax.experimental.pallas import tpu as pltpu
from jax.experimental.pallas import tpu_sc as plsc

D = 64
L = 16
NC, NS = 2, 16
NW = NC * NS
C = 128
DH, DL = D // 8, 8
NB = C // 128


def _lane_shuffle(v, idx):
    return lax.gather(
        v, idx[:, None],
        lax.GatherDimensionNumbers(
            offset_dims=(), collapsed_slice_dims=(0,), start_index_map=(0,)),
        (1,), mode=lax.GatherScatterMode.PROMISE_IN_BOUNDS)


def _lane_sum(v):
    lanes = lax.iota(jnp.int32, L)
    for k in (8, 4, 2, 1):
        v = v + _lane_shuffle(v, lanes ^ jnp.int32(k))
    return v


def _rsqrt_vec(x):
    i = lax.bitcast_convert_type(x, jnp.int32)
    i = jnp.int32(0x5F3759DF) - lax.shift_right_arithmetic(i, jnp.int32(1))
    y = lax.bitcast_convert_type(i, jnp.float32)
    for _ in range(2):
        y = y * (jnp.float32(1.5) - jnp.float32(0.5) * x * y * y)
    return y


def _make_sc_call(seq, batch):
    n_rows = seq * batch
    n_chunks_total = n_rows // C
    assert n_chunks_total % (4 * NW) == 0
    chunks_per_w = n_chunks_total // NW
    bt_per_s = batch // C
    mesh = plsc.VectorSubcoreMesh(core_axis_name="c", subcore_axis_name="s")
    NBUF = 4

    @functools.partial(
        pl.kernel,
        mesh=mesh,
        compiler_params=pltpu.CompilerParams(use_tc_tiling_on_sc=False,
                                             needs_layout_passes=False),
        out_type=jax.ShapeDtypeStruct((seq, DH, batch // 128, DL, 128),
                                      jnp.float32),
        scratch_types=(
            [pltpu.VMEM((C,), jnp.int32)] * NBUF
            + [pltpu.VMEM((C, D), jnp.float32)] * NBUF
            + [pltpu.VMEM((DH, DL, 129), jnp.float32)] * NBUF
            + [pltpu.VMEM((D,), jnp.float32)] * 2
            + [pltpu.SemaphoreType.DMA] * (3 * NBUF)
        ),
    )
    def sc_fn(idx_hbm, table_hbm, gamma_hbm, beta_hbm, out_hbm, *bufs):
        wid = lax.axis_index("s") * NC + lax.axis_index("c")
        g_base = wid * chunks_per_w

        idx_v = bufs[0:NBUF]
        rows_v = bufs[NBUF:2 * NBUF]
        outt_v = bufs[2 * NBUF:3 * NBUF]
        gv, bv = bufs[3 * NBUF:3 * NBUF + 2]
        isem = bufs[3 * NBUF + 2:3 * NBUF + 2 + NBUF]
        gsem = bufs[3 * NBUF + 2 + NBUF:3 * NBUF + 2 + 2 * NBUF]
        osem = bufs[3 * NBUF + 2 + 2 * NBUF:3 * NBUF + 2 + 3 * NBUF]

        pltpu.sync_copy(gamma_hbm, gv)
        pltpu.sync_copy(beta_hbm, bv)
        gslab = [gv[pl.ds(j * L, L)] for j in range(D // L)]
        bslab = [bv[pl.ds(j * L, L)] for j in range(D // L)]

        def fire_idx(g, slot):
            pltpu.async_copy(idx_hbm.at[pl.ds(g * C, C)], idx_v[slot],
                             isem[slot])

        def wait_idx(slot):
            pltpu.make_async_copy(idx_hbm.at[pl.ds(0, C)], idx_v[slot],
                                  isem[slot]).wait()

        def fire_gather(slot):
            for j in range(C // L):
                sl = pl.ds(j * L, L)
                iv = lax.shift_left(idx_v[slot][sl], jnp.int32(1))
                pltpu.async_copy(table_hbm.at[iv], rows_v[slot].at[sl],
                                 gsem[slot])

        def wait_gather(slot):
            pltpu.make_async_copy(table_hbm.at[pl.ds(0, C)], rows_v[slot],
                                  gsem[slot]).wait()

        def fire_out(g, slot):
            s_i = g // bt_per_s
            bt_i = lax.rem(g, jnp.int32(bt_per_s))
            for dh in range(DH):
                pltpu.async_copy(outt_v[slot].at[dh, :, pl.ds(0, 128)],
                                 out_hbm.at[s_i, dh, bt_i], osem[slot])

        def wait_out(slot):
            for dh in range(DH):
                pltpu.make_async_copy(outt_v[slot].at[dh, :, pl.ds(0, 128)],
                                      out_hbm.at[0, dh, 0],
                                      osem[slot]).wait()

        lanes = lax.iota(jnp.int32, L)
        i0s = [jnp.int32(2 * j) + lax.shift_right_logical(lanes, jnp.int32(3))
               for j in range(D // L)]
        i1 = jnp.bitwise_and(lanes, jnp.int32(7))

        def compute_chunk(rows, outt):
            def row_body(r, carry):
                vs = [rows[r, pl.ds(j * L, L)] for j in range(D // L)]
                s = (vs[0] + vs[1]) + (vs[2] + vs[3])
                ss = ((vs[0] * vs[0] + vs[1] * vs[1])
                      + (vs[2] * vs[2] + vs[3] * vs[3]))
                mean = _lane_sum(s) * jnp.float32(1.0 / D)
                var = _lane_sum(ss) * jnp.float32(1.0 / D) - mean * mean
                scale = _rsqrt_vec(var + jnp.float32(1e-5))
                rsplat = jnp.full((L,), r, jnp.int32)
                for j in range(D // L):
                    w = (vs[j] - mean) * scale * gslab[j] + bslab[j]
                    plsc.store_scatter(outt, [i0s[j], i1, rsplat], w)
                return carry

            lax.fori_loop(0, C, row_body, 0, unroll=2)

        pltpu.sync_copy(idx_hbm.at[pl.ds(g_base * C, C)], idx_v[0])
        fire_gather(0)
        for k in range(1, NBUF):
            fire_idx(g_base + k, k)
        for k in range(1, NBUF - 1):
            wait_idx(k)
            fire_gather(k)

        def quad_body(p, carry):
            for off in range(NBUF):
                slot = off
                g = NBUF * p + off
                wait_gather(slot)

                @pl.when(g + NBUF < chunks_per_w)
                def _():
                    fire_idx(g_base + g + NBUF, slot)

                @pl.when(g >= NBUF)
                def _():
                    wait_out(slot)

                @pl.when(g + NBUF - 1 < chunks_per_w)
                def _():
                    pslot = (off + NBUF - 1) % NBUF
                    wait_idx(pslot)
                    fire_gather(pslot)

                compute_chunk(rows_v[slot], outt_v[slot])
                fire_out(g_base + g, slot)
            return carry

        lax.fori_loop(0, chunks_per_w // NBUF, quad_body, 0)
        for k in range(NBUF):
            wait_out(k)

    return sc_fn


def kernel(x, table, gamma, beta):
    b, s = x.shape
    v, d = table.shape
    assert d == D
    xt = x.T.reshape(-1).astype(jnp.int32)
    tablep = jnp.pad(table, ((0, 0), (0, d))).reshape(2 * v, d)
    out5 = _make_sc_call(s, b)(xt, tablep, gamma, beta)
    out = jnp.transpose(out5, (2, 4, 0, 1, 3)).reshape(b, s, d)
    return out
